# baseline (speedup 1.0000x reference)
import jax
import jax.numpy as jnp
from jax.experimental import pallas as pl
from jax.experimental.pallas import tpu as pltpu

_RANK = 1024
_N_OUT = 8
_N_PROC = 32
_K = 3
_BLK = 4096


_SUB = 1024


def _router_kernel(x_ref, wo_ref, wp_ref, ow_ref, pi_ref):
    dn = (((1,), (1,)), ((), ()))
    wo = wo_ref[...]
    wp = wp_ref[...]
    iota = jax.lax.broadcasted_iota(jnp.int32, (_N_PROC, _SUB), 0)
    for c in range(_BLK // _SUB):
        sl = pl.ds(c * _SUB, _SUB)
        xb = x_ref[sl, :]
        so = jax.lax.dot_general(wo, xb, dn,
                                 preferred_element_type=jnp.float32)
        sp = jax.lax.dot_general(wp, xb, dn,
                                 preferred_element_type=jnp.float32)

        m = jnp.max(so, axis=0, keepdims=True)
        e = jnp.exp(so - m)
        ow_ref[:, sl] = e / jnp.sum(e, axis=0, keepdims=True)

        s = sp
        for j in range(_K):
            mx = jnp.max(s, axis=0, keepdims=True)
            idx = jnp.min(jnp.where(s >= mx, iota, _N_PROC),
                          axis=0, keepdims=True)
            pi_ref[j:j + 1, sl] = idx
            s = jnp.where(iota == idx, -jnp.inf, s)


@jax.jit
def kernel(x, W_out, W_proc):
    B, S, R = x.shape
    n_tok = B * S
    xf = x.reshape(n_tok, R)
    grid = (n_tok // _BLK,)
    ow_t, pi_t = pl.pallas_call(
        _router_kernel,
        grid=grid,
        in_specs=[
            pl.BlockSpec((_BLK, R), lambda i: (i, 0)),
            pl.BlockSpec((_N_OUT, R), lambda i: (0, 0)),
            pl.BlockSpec((_N_PROC, R), lambda i: (0, 0)),
        ],
        out_specs=[
            pl.BlockSpec((_N_OUT, _BLK), lambda i: (0, i)),
            pl.BlockSpec((_K, _BLK), lambda i: (0, i)),
        ],
        out_shape=[
            jax.ShapeDtypeStruct((_N_OUT, n_tok), jnp.float32),
            jax.ShapeDtypeStruct((_K, n_tok), jnp.int32),
        ],
        compiler_params=pltpu.CompilerParams(
            dimension_semantics=("parallel",),
        ),
    )(xf, W_out, W_proc)
    ow = ow_t.T.reshape(B, S, _N_OUT)
    pi = pi_t.T.reshape(B, S, _K)
    return ow, pi


# speedup vs baseline: 1.0272x; 1.0272x over previous
import jax
import jax.numpy as jnp
from jax.experimental import pallas as pl
from jax.experimental.pallas import tpu as pltpu

_RANK = 1024
_N_OUT = 8
_N_PROC = 32
_K = 3
_BLK = 4096
_SUB = 1024


def _router_kernel(x_ref, w_ref, ow_ref, pi_ref):
    dn = (((1,), (1,)), ((), ()))
    w = w_ref[...]
    iota = jax.lax.broadcasted_iota(jnp.int32, (_N_PROC, _SUB), 0)
    for c in range(_BLK // _SUB):
        sl = pl.ds(c * _SUB, _SUB)
        xb = x_ref[sl, :]
        st = jax.lax.dot_general(w, xb, dn,
                                 preferred_element_type=jnp.float32)
        so = st[:_N_OUT, :]
        sp = st[_N_OUT:, :]

        m = jnp.max(so, axis=0, keepdims=True)
        e = jnp.exp(so - m)
        ow_ref[:, sl] = e / jnp.sum(e, axis=0, keepdims=True)

        s = sp
        for j in range(_K):
            mx = jnp.max(s, axis=0, keepdims=True)
            idx = jnp.min(jnp.where(s >= mx, iota, _N_PROC),
                          axis=0, keepdims=True)
            pi_ref[j:j + 1, sl] = idx
            s = jnp.where(iota == idx, -jnp.inf, s)


@jax.jit
def kernel(x, W_out, W_proc):
    B, S, R = x.shape
    n_tok = B * S
    xf = x.reshape(n_tok, R)
    w_all = jnp.concatenate([W_out, W_proc], axis=0)
    grid = (n_tok // _BLK,)
    ow_t, pi_t = pl.pallas_call(
        _router_kernel,
        grid=grid,
        in_specs=[
            pl.BlockSpec((_BLK, R), lambda i: (i, 0)),
            pl.BlockSpec((_N_OUT + _N_PROC, R), lambda i: (0, 0)),
        ],
        out_specs=[
            pl.BlockSpec((_N_OUT, _BLK), lambda i: (0, i)),
            pl.BlockSpec((_K, _BLK), lambda i: (0, i)),
        ],
        out_shape=[
            jax.ShapeDtypeStruct((_N_OUT, n_tok), jnp.float32),
            jax.ShapeDtypeStruct((_K, n_tok), jnp.int32),
        ],
        compiler_params=pltpu.CompilerParams(
            dimension_semantics=("parallel",),
        ),
    )(xf, w_all)
    ow = ow_t.T.reshape(B, S, _N_OUT)
    pi = pi_t.T.reshape(B, S, _K)
    return ow, pi
